# baseline (speedup 1.0000x reference)
import functools

import jax
import jax.numpy as jnp
from jax import lax
from jax.experimental import pallas as pl
from jax.experimental.pallas import tpu as pltpu
from jax.experimental.pallas import tpu_sc as plsc

_NC = 2
_NS = 16
_NW = _NC * _NS
_LANES = 16


def _body(x_hbm, pe_hbm, out_hbm, buf, pe_v, *, L, D, seg, rows_w, nblk, blk):
    cid = lax.axis_index("c")
    sid = lax.axis_index("s")
    wid = sid * _NC + cid
    pltpu.sync_copy(pe_hbm.at[pl.ds(0, 3 * D)], pe_v)
    wbase = wid * rows_w
    last_base = rows_w - blk

    def block(i, carry):
        rbase = jnp.minimum(i * blk, last_base)
        g0 = wbase + rbase
        w0 = g0 * D
        pltpu.sync_copy(x_hbm.at[pl.ds(w0, blk * D)], buf)

        def row(r, c2):
            pos = lax.rem(g0 + r, L)
            off = jnp.where(pos == 0, 0, jnp.where(pos <= seg, D, 2 * D))
            rb = r * D
            for d in range(D // _LANES):
                sl = pl.ds(rb + d * _LANES, _LANES)
                buf[sl] = buf[sl] + pe_v[pl.ds(off + d * _LANES, _LANES)]
            return c2

        lax.fori_loop(0, blk, row, 0)
        pltpu.sync_copy(buf, out_hbm.at[pl.ds(w0, blk * D)])
        return carry

    lax.fori_loop(0, nblk, block, 0)


def kernel(x, pe):
    B, L, D = x.shape
    seg = (L - 1) // 2
    rows = B * L
    rows_w = rows // _NW
    blk = 48
    nblk = -(-rows_w // blk)

    body = functools.partial(
        _body, L=L, D=D, seg=seg, rows_w=rows_w, nblk=nblk, blk=blk)
    mesh = plsc.VectorSubcoreMesh(
        core_axis_name="c", subcore_axis_name="s",
        num_cores=_NC, num_subcores=_NS)
    out = pl.kernel(
        body,
        out_type=jax.ShapeDtypeStruct((rows * D,), jnp.float32),
        mesh=mesh,
        scratch_types=[
            pltpu.VMEM((blk * D,), jnp.float32),
            pltpu.VMEM((3 * D,), jnp.float32),
        ],
    )(x.reshape(-1), pe.reshape(-1))
    return out.reshape(x.shape)


# speedup vs baseline: 1.2757x; 1.2757x over previous
import functools

import jax
import jax.numpy as jnp
from jax import lax
from jax.experimental import pallas as pl
from jax.experimental.pallas import tpu as pltpu
from jax.experimental.pallas import tpu_sc as plsc

_NC = 2
_NS = 16
_NW = _NC * _NS
_LANES = 16
_BLK = 48


def _body(x_hbm, pe_hbm, out_hbm, buf, pe_v, *, L, D, seg, nblk_w):
    cid = lax.axis_index("c")
    sid = lax.axis_index("s")
    wid = sid * _NC + cid
    nd = D // _LANES
    pltpu.sync_copy(pe_hbm.at[pl.ds(0, 3 * D)], pe_v)

    jobs_w = nblk_w * _BLK // seg

    def block(t, carry):
        job = wid * jobs_w + t // (seg // _BLK)
        jb = lax.rem(t, seg // _BLK)
        b = job // 2
        s = lax.rem(job, 2)
        row0 = b * L + 1 + s * seg + jb * _BLK
        w0 = row0 * D
        off = (1 + s) * D
        pltpu.sync_copy(x_hbm.at[pl.ds(w0, _BLK * D)], buf)
        for d in range(nd):
            pev = pe_v[pl.ds(off + d * _LANES, _LANES)]

            def rloop(r, c2, _d=d, _pev=pev):
                sl = pl.ds(r * D + _d * _LANES, _LANES)
                buf[sl] = buf[sl] + _pev
                return c2

            lax.fori_loop(0, _BLK, rloop, 0, unroll=8)
        pltpu.sync_copy(buf, out_hbm.at[pl.ds(w0, _BLK * D)])
        return carry

    lax.fori_loop(0, nblk_w, block, 0)

    def cls_block(j, carry):
        row = (wid + j * _NW) * L
        w0 = row * D
        pltpu.sync_copy(x_hbm.at[pl.ds(w0, D)], buf.at[pl.ds(0, D)])
        for d in range(nd):
            sl = pl.ds(d * _LANES, _LANES)
            buf[sl] = buf[sl] + pe_v[pl.ds(d * _LANES, _LANES)]
        pltpu.sync_copy(buf.at[pl.ds(0, D)], out_hbm.at[pl.ds(w0, D)])
        return carry

    lax.fori_loop(0, 2, cls_block, 0)


def kernel(x, pe):
    B, L, D = x.shape
    seg = (L - 1) // 2
    rows = B * L
    nblk_w = (rows - B) // (_NW * _BLK)

    body = functools.partial(_body, L=L, D=D, seg=seg, nblk_w=nblk_w)
    mesh = plsc.VectorSubcoreMesh(
        core_axis_name="c", subcore_axis_name="s",
        num_cores=_NC, num_subcores=_NS)
    out = pl.kernel(
        body,
        out_type=jax.ShapeDtypeStruct((rows * D,), jnp.float32),
        mesh=mesh,
        scratch_types=[
            pltpu.VMEM((_BLK * D,), jnp.float32),
            pltpu.VMEM((3 * D,), jnp.float32),
        ],
    )(x.reshape(-1), pe.reshape(-1))
    return out.reshape(x.shape)
